# baseline (speedup 1.0000x reference)
import functools

import jax
import jax.numpy as jnp
from jax import lax
from jax.experimental import pallas as pl
from jax.experimental.pallas import tpu as pltpu
from jax.experimental.pallas import tpu_sc as plsc

_SIZE = 1_000_000
_DIM = 32
_BATCH = 4096
_NSAMP = 50
_NCAND = _NSAMP - 1
_EPS = 1e-5

_G = 4
_NSLOT = _SIZE // _G
_NC, _NS, _L = 2, 16, 16
_NW = _NC * _NS

_SBW = 512
_NSB = _SIZE // _SBW
_SBREM = _SIZE - _NSB * _SBW
_SB_PER_W = 31


_PITCH = _SBW + 9
_PPITCH = _SBREM + 9


def _sc_repack_body(wt_hbm, out_hbm, in0, in1, tr0, tr1, sk, in_p, sem_in, sem_out):
    wid = lax.axis_index("s") * _NC + lax.axis_index("c")
    iota = lax.iota(jnp.int32, _L)
    ins = (in0, in1)
    trs = (tr0, tr1)
    iota_p_lo = iota * _PITCH
    iota_p_hi = (iota + _L) * _PITCH

    def in_copies(k, buf):
        sb = wid + _NW * k
        return [
            pltpu.make_async_copy(
                wt_hbm.at[:, pl.ds(sb * _SBW, _SBW)], buf, sem_in
            )
        ]

    def out_copy(k, buf):
        sb = wid + _NW * k
        return pltpu.make_async_copy(
            buf, out_hbm.at[pl.ds(sb * (_SBW // _G), _SBW // _G), :], sem_out
        )

    def valid(k):
        return (wid + _NW * k) < _NSB

    @pl.when(valid(0))
    def _():
        for h in in_copies(0, in0):
            h.start()

    def do_pair(k2, carry):
        for par in (0, 1):
            k = 2 * k2 + par
            buf = ins[par]
            tr = trs[par]
            v_k = valid(k)

            @pl.when(v_k)
            def _(k=k, buf=buf):
                for h in in_copies(k, buf):
                    h.wait()

            @pl.when(valid(k + 1))
            def _(k=k, par=par):
                for h in in_copies(k + 1, ins[1 - par]):
                    h.start()

            @pl.when(v_k & (k >= 2))
            def _(k=k, tr=tr):
                out_copy(k - 2, tr).wait()

            @pl.when(v_k)
            def _(buf=buf, tr=tr):
                def do_skew(r4, inner):
                    for v in range(4):
                        r = r4 * 4 + v
                        for q in range(_SBW // _L):
                            sk[pl.ds(r * _PITCH + q * _L, _L)] = buf[
                                r, pl.ds(q * _L, _L)
                            ]
                    return inner

                lax.fori_loop(0, _DIM // 4, do_skew, 0)

                def do_s(s8, inner):
                    for u in range(8):
                        s = s8 * 8 + u
                        for ck in range(8):
                            base = jnp.int32(s * _G + ck // 2)
                            idx = (iota_p_hi if ck % 2 else iota_p_lo) + base
                            tr[s, pl.ds(ck * _L, _L)] = plsc.load_gather(
                                sk, [idx]
                            )
                    return inner

                lax.fori_loop(0, (_SBW // _G) // 8, do_s, 0)

            @pl.when(v_k)
            def _(k=k, tr=tr):
                out_copy(k, tr).start()

        return carry

    lax.fori_loop(0, _SB_PER_W, do_pair, 0)

    k_last = (_NSB - 1 - wid) // _NW
    for par in (0, 1):
        klp = k_last - ((k_last - par) % 2)

        @pl.when(klp >= 0)
        def _(klp=klp, par=par):
            out_copy(klp, trs[par]).wait()

    @pl.when(wid == 1)
    def _():
        pltpu.sync_copy(wt_hbm.at[:, pl.ds(_NSB * _SBW, _SBREM)], in_p)

        def do_skew(r, inner):
            for q in range(_SBREM // _L):
                sk[pl.ds(r * _PPITCH + q * _L, _L)] = in_p[r, pl.ds(q * _L, _L)]
            return inner

        lax.fori_loop(0, _DIM, do_skew, 0)

        def do_s(s, inner):
            for ck in range(8):
                base = s * _G + ck // 2
                idx = (iota + (_L if ck % 2 else 0)) * _PPITCH + base
                tr0[s, pl.ds(ck * _L, _L)] = plsc.load_gather(sk, [idx])
            return inner

        lax.fori_loop(0, _SBREM // _G, do_s, 0)
        pltpu.sync_copy(
            tr0.at[pl.ds(0, _SBREM // _G), :],
            out_hbm.at[pl.ds(_NSB * (_SBW // _G), _SBREM // _G), :],
        )


_sc_repack = functools.partial(
    pl.kernel,
    out_type=jax.ShapeDtypeStruct((_NSLOT, _G * _DIM), jnp.float32),
    mesh=plsc.VectorSubcoreMesh(
        core_axis_name="c", subcore_axis_name="s", num_cores=_NC, num_subcores=_NS
    ),
    scratch_types=[
        pltpu.VMEM((_DIM, _SBW), jnp.float32),
        pltpu.VMEM((_DIM, _SBW), jnp.float32),
        pltpu.VMEM((_SBW // _G, 128), jnp.float32),
        pltpu.VMEM((_SBW // _G, 128), jnp.float32),
        pltpu.VMEM((_DIM * _PITCH,), jnp.float32),
        pltpu.VMEM((_DIM, _SBREM), jnp.float32),
        pltpu.SemaphoreType.DMA,
        pltpu.SemaphoreType.DMA,
    ],
    compiler_params=pltpu.CompilerParams(
        needs_layout_passes=False, use_tc_tiling_on_sc=True
    ),
)(_sc_repack_body)


_ROWS_W = _BATCH // _NW
_CHUNK = 16
_NCHUNK = _ROWS_W // _CHUNK
_NIDX = _CHUNK * _NSAMP
_GROUPS = 4
_OUTP = _GROUPS * _L

_GATHER_SPLITS = [128] * (_NIDX // 128) + ([_NIDX % 128] if _NIDX % 128 else [])


def _sc_body(idx_hbm, w_hbm, x_hbm, idx_v, rows_v, out_v, sem):
    wid = lax.axis_index("s") * _NC + lax.axis_index("c")
    iota = lax.iota(jnp.int32, _L)

    def do_chunk(c, carry):
        row0 = wid * _ROWS_W + c * _CHUNK
        flat0 = row0 * _NSAMP
        pltpu.sync_copy(idx_hbm.at[pl.ds(flat0, _NIDX)], idx_v)

        handles = []
        off = 0
        for sz in _GATHER_SPLITS:
            handles.append(
                pltpu.async_copy(
                    w_hbm.at[idx_v.at[pl.ds(off, sz)]],
                    rows_v.at[pl.ds(off, sz), :],
                    sem,
                )
            )
            off += sz
        for h in handles:
            h.wait()

        def do_row(b, inner):
            rbase = b * _NSAMP
            ridx = []
            for g in range(_GROUPS):
                r = rbase + 1 + g * _L + iota
                if g == _GROUPS - 1:
                    r = jnp.minimum(r, rbase + _NCAND)
                ridx.append(r)
            s_lo = rows_v[rbase, pl.ds(0, _L)]
            s_hi = rows_v[rbase, pl.ds(_L, _L)]
            accs = [None] * _GROUPS
            for d in range(_DIM):
                s = s_lo[d] if d < _L else s_hi[d - _L]
                cv = jnp.full((_L,), s, jnp.float32)
                col = jnp.full((_L,), d, jnp.int32)
                for g in range(_GROUPS):
                    gv = plsc.load_gather(rows_v, [ridx[g], col])
                    if d == 0:
                        accs[g] = gv * cv
                    else:
                        accs[g] = accs[g] - gv * cv
            for g in range(_GROUPS):
                out_v[b, pl.ds(g * _L, _L)] = accs[g]
            return inner

        lax.fori_loop(0, _CHUNK, do_row, 0)
        pltpu.sync_copy(out_v, x_hbm.at[pl.ds(row0, _CHUNK), :])
        return carry

    lax.fori_loop(0, _NCHUNK, do_chunk, 0)


_sc_gather_dot = functools.partial(
    pl.kernel,
    out_type=jax.ShapeDtypeStruct((_BATCH, _OUTP), jnp.float32),
    mesh=plsc.VectorSubcoreMesh(
        core_axis_name="c", subcore_axis_name="s", num_cores=_NC, num_subcores=_NS
    ),
    scratch_types=[
        pltpu.VMEM((_NIDX,), jnp.int32),
        pltpu.VMEM((_NIDX, _DIM), jnp.float32),
        pltpu.VMEM((_CHUNK, _OUTP), jnp.float32),
        pltpu.SemaphoreType.DMA,
    ],
    compiler_params=pltpu.CompilerParams(
        needs_layout_passes=False, use_tc_tiling_on_sc=False
    ),
)(_sc_body)


def _tc_finish_body(x_ref, o_ref):
    x = x_ref[...][:, :_NCAND]
    x = jnp.maximum(x, 1.0 + _EPS)
    o_ref[...] = -jnp.log(x + jnp.sqrt((x - 1.0) * (x + 1.0)))


def _tc_finish(x):
    blk = 512
    return pl.pallas_call(
        _tc_finish_body,
        grid=(_BATCH // blk,),
        in_specs=[pl.BlockSpec((blk, _OUTP), lambda i: (i, 0))],
        out_specs=pl.BlockSpec((blk, _NCAND), lambda i: (i, 0)),
        out_shape=jax.ShapeDtypeStruct((_BATCH, _NCAND), jnp.float32),
    )(x)


def kernel(inputs, weight):
    idx_flat = inputs.reshape(-1)
    w4 = _sc_repack(weight.T)
    wrows = w4.reshape(_SIZE, _DIM)
    x = _sc_gather_dot(idx_flat, wrows)
    return _tc_finish(x)


# speedup vs baseline: 1.0200x; 1.0200x over previous
import functools

import jax
import jax.numpy as jnp
from jax import lax
from jax.experimental import pallas as pl
from jax.experimental.pallas import tpu as pltpu
from jax.experimental.pallas import tpu_sc as plsc

_SIZE = 1_000_000
_DIM = 32
_BATCH = 4096
_NSAMP = 50
_NCAND = _NSAMP - 1
_EPS = 1e-5

_G = 4
_NSLOT = _SIZE // _G
_NC, _NS, _L = 2, 16, 16
_NW = _NC * _NS

_SBW = 512
_NSB = _SIZE // _SBW
_SBREM = _SIZE - _NSB * _SBW
_SB_PER_W = 31


_PITCH = _SBW + 9
_PPITCH = _SBREM + 9


def _sc_repack_body(wt_hbm, out_hbm, in0, in1, tr0, tr1, sk, in_p, sem_in, sem_out):
    wid = lax.axis_index("s") * _NC + lax.axis_index("c")
    iota = lax.iota(jnp.int32, _L)
    ins = (in0, in1)
    trs = (tr0, tr1)
    iota_p_lo = iota * _PITCH
    iota_p_hi = (iota + _L) * _PITCH

    def in_copies(k, buf):
        sb = wid + _NW * k
        return [
            pltpu.make_async_copy(
                wt_hbm.at[:, pl.ds(sb * _SBW, _SBW)], buf, sem_in
            )
        ]

    def out_copy(k, buf):
        sb = wid + _NW * k
        return pltpu.make_async_copy(
            buf, out_hbm.at[pl.ds(sb * (_SBW // _G), _SBW // _G), :], sem_out
        )

    def valid(k):
        return (wid + _NW * k) < _NSB

    @pl.when(valid(0))
    def _():
        for h in in_copies(0, in0):
            h.start()

    def do_pair(k2, carry):
        for par in (0, 1):
            k = 2 * k2 + par
            buf = ins[par]
            tr = trs[par]
            v_k = valid(k)

            @pl.when(v_k)
            def _(k=k, buf=buf):
                for h in in_copies(k, buf):
                    h.wait()

            @pl.when(valid(k + 1))
            def _(k=k, par=par):
                for h in in_copies(k + 1, ins[1 - par]):
                    h.start()

            @pl.when(v_k & (k >= 2))
            def _(k=k, tr=tr):
                out_copy(k - 2, tr).wait()

            @pl.when(v_k)
            def _(buf=buf, tr=tr):
                def do_skew(r, inner):
                    for q in range(_SBW // _L):
                        sk[pl.ds(r * _PITCH + q * _L, _L)] = buf[
                            r, pl.ds(q * _L, _L)
                        ]
                    return inner

                lax.fori_loop(0, _DIM, do_skew, 0)

                def do_s(s4, inner):
                    for u in range(4):
                        s = s4 * 4 + u
                        for ck in range(8):
                            base = jnp.int32(s * _G + ck // 2)
                            idx = (iota_p_hi if ck % 2 else iota_p_lo) + base
                            tr[s, pl.ds(ck * _L, _L)] = plsc.load_gather(
                                sk, [idx]
                            )
                    return inner

                lax.fori_loop(0, (_SBW // _G) // 4, do_s, 0)

            @pl.when(v_k)
            def _(k=k, tr=tr):
                out_copy(k, tr).start()

        return carry

    lax.fori_loop(0, _SB_PER_W, do_pair, 0)

    k_last = (_NSB - 1 - wid) // _NW
    for par in (0, 1):
        klp = k_last - ((k_last - par) % 2)

        @pl.when(klp >= 0)
        def _(klp=klp, par=par):
            out_copy(klp, trs[par]).wait()

    @pl.when(wid == 1)
    def _():
        pltpu.sync_copy(wt_hbm.at[:, pl.ds(_NSB * _SBW, _SBREM)], in_p)

        def do_skew(r, inner):
            for q in range(_SBREM // _L):
                sk[pl.ds(r * _PPITCH + q * _L, _L)] = in_p[r, pl.ds(q * _L, _L)]
            return inner

        lax.fori_loop(0, _DIM, do_skew, 0)

        def do_s(s, inner):
            for ck in range(8):
                base = s * _G + ck // 2
                idx = (iota + (_L if ck % 2 else 0)) * _PPITCH + base
                tr0[s, pl.ds(ck * _L, _L)] = plsc.load_gather(sk, [idx])
            return inner

        lax.fori_loop(0, _SBREM // _G, do_s, 0)
        pltpu.sync_copy(
            tr0.at[pl.ds(0, _SBREM // _G), :],
            out_hbm.at[pl.ds(_NSB * (_SBW // _G), _SBREM // _G), :],
        )


_sc_repack = functools.partial(
    pl.kernel,
    out_type=jax.ShapeDtypeStruct((_NSLOT, _G * _DIM), jnp.float32),
    mesh=plsc.VectorSubcoreMesh(
        core_axis_name="c", subcore_axis_name="s", num_cores=_NC, num_subcores=_NS
    ),
    scratch_types=[
        pltpu.VMEM((_DIM, _SBW), jnp.float32),
        pltpu.VMEM((_DIM, _SBW), jnp.float32),
        pltpu.VMEM((_SBW // _G, 128), jnp.float32),
        pltpu.VMEM((_SBW // _G, 128), jnp.float32),
        pltpu.VMEM((_DIM * _PITCH,), jnp.float32),
        pltpu.VMEM((_DIM, _SBREM), jnp.float32),
        pltpu.SemaphoreType.DMA,
        pltpu.SemaphoreType.DMA,
    ],
    compiler_params=pltpu.CompilerParams(
        needs_layout_passes=False, use_tc_tiling_on_sc=True
    ),
)(_sc_repack_body)


_ROWS_W = _BATCH // _NW
_CHUNK = 16
_NCHUNK = _ROWS_W // _CHUNK
_NIDX = _CHUNK * _NSAMP
_GROUPS = 4
_OUTP = _GROUPS * _L

_GATHER_SPLITS = [128] * (_NIDX // 128) + ([_NIDX % 128] if _NIDX % 128 else [])


def _sc_body(idx_hbm, w_hbm, x_hbm, idx0, idx1, rows0, rows1, out_v, sem0, sem1):
    wid = lax.axis_index("s") * _NC + lax.axis_index("c")
    iota = lax.iota(jnp.int32, _L)
    idxs = (idx0, idx1)
    rows = (rows0, rows1)
    sems = (sem0, sem1)

    def fetch(c):
        par = c % 2
        idx_v = idxs[par]
        flat0 = (wid * _ROWS_W + c * _CHUNK) * _NSAMP
        pltpu.sync_copy(idx_hbm.at[pl.ds(flat0, _NIDX)], idx_v)
        handles = []
        off = 0
        for sz in _GATHER_SPLITS:
            handles.append(
                pltpu.async_copy(
                    w_hbm.at[idx_v.at[pl.ds(off, sz)]],
                    rows[par].at[pl.ds(off, sz), :],
                    sem0 if par == 0 else sem1,
                )
            )
            off += sz
        return handles

    pending = fetch(0)
    for c in range(_NCHUNK):
        for h in pending:
            h.wait()
        rows_v = rows[c % 2]
        if c + 1 < _NCHUNK:
            pending = fetch(c + 1)
        row0 = wid * _ROWS_W + c * _CHUNK

        def do_row(b, inner, rows_v=rows_v):
            rbase = b * _NSAMP
            ridx = []
            for g in range(_GROUPS):
                r = rbase + 1 + g * _L + iota
                if g == _GROUPS - 1:
                    r = jnp.minimum(r, rbase + _NCAND)
                ridx.append(r)
            s_lo = rows_v[rbase, pl.ds(0, _L)]
            s_hi = rows_v[rbase, pl.ds(_L, _L)]
            accs = [None] * _GROUPS
            for d in range(_DIM):
                s = s_lo[d] if d < _L else s_hi[d - _L]
                cv = jnp.full((_L,), s, jnp.float32)
                col = jnp.full((_L,), d, jnp.int32)
                for g in range(_GROUPS):
                    gv = plsc.load_gather(rows_v, [ridx[g], col])
                    if d == 0:
                        accs[g] = gv * cv
                    else:
                        accs[g] = accs[g] - gv * cv
            for g in range(_GROUPS):
                out_v[b, pl.ds(g * _L, _L)] = accs[g]
            return inner

        lax.fori_loop(0, _CHUNK, do_row, 0)
        pltpu.sync_copy(out_v, x_hbm.at[pl.ds(row0, _CHUNK), :])


_sc_gather_dot = functools.partial(
    pl.kernel,
    out_type=jax.ShapeDtypeStruct((_BATCH, _OUTP), jnp.float32),
    mesh=plsc.VectorSubcoreMesh(
        core_axis_name="c", subcore_axis_name="s", num_cores=_NC, num_subcores=_NS
    ),
    scratch_types=[
        pltpu.VMEM((_NIDX,), jnp.int32),
        pltpu.VMEM((_NIDX,), jnp.int32),
        pltpu.VMEM((_NIDX, _DIM), jnp.float32),
        pltpu.VMEM((_NIDX, _DIM), jnp.float32),
        pltpu.VMEM((_CHUNK, _OUTP), jnp.float32),
        pltpu.SemaphoreType.DMA,
        pltpu.SemaphoreType.DMA,
    ],
    compiler_params=pltpu.CompilerParams(
        needs_layout_passes=False, use_tc_tiling_on_sc=False
    ),
)(_sc_body)


def _tc_finish_body(x_ref, o_ref):
    x = x_ref[...][:, :_NCAND]
    x = jnp.maximum(x, 1.0 + _EPS)
    o_ref[...] = -jnp.log(x + jnp.sqrt((x - 1.0) * (x + 1.0)))


def _tc_finish(x):
    blk = 512
    return pl.pallas_call(
        _tc_finish_body,
        grid=(_BATCH // blk,),
        in_specs=[pl.BlockSpec((blk, _OUTP), lambda i: (i, 0))],
        out_specs=pl.BlockSpec((blk, _NCAND), lambda i: (i, 0)),
        out_shape=jax.ShapeDtypeStruct((_BATCH, _NCAND), jnp.float32),
    )(x)


def kernel(inputs, weight):
    idx_flat = inputs.reshape(-1)
    w4 = _sc_repack(weight.T)
    wrows = w4.reshape(_SIZE, _DIM)
    x = _sc_gather_dot(idx_flat, wrows)
    return _tc_finish(x)
